# baseline (speedup 1.0000x reference)
import functools

import jax
import jax.numpy as jnp
import numpy as np
from jax.experimental import pallas as pl
from jax.experimental.pallas import tpu as pltpu

_B = 128
_A = 64
_EMB = 32
_POS_EMB = 16
_ENC_DIM = 128
_L = 4
_G = 16
_JB = 8


def _dotT(x, w):
    return jax.lax.dot_general(x, w, (((1,), (1,)), ((), ())),
                               preferred_element_type=jnp.float32)


def _body(*refs):
    (pos_ref, tf_ref, enc_ref, pemb_ref, na_ref,
     fc1W_ref, fc1b_ref, fc2W_ref, fc2b_ref, linW_ref, linb_ref) = refs[:11]
    layer_refs = refs[11:11 + 8 * _L]
    Esel_ref, TileEye_ref, Q_ref, F_ref, out_ref = refs[11 + 8 * _L:]

    G, A, EMB, JB = _G, _A, _EMB, _JB
    NJ = A // JB
    W = JB * EMB

    f32 = jnp.float32
    bf16 = jnp.bfloat16
    dot = functools.partial(jnp.dot, preferred_element_type=f32)

    enc = enc_ref[...]
    na = na_ref[...]
    e1 = jnp.tanh(_dotT(enc, fc1W_ref[...]) + fc1b_ref[...])
    enc_emb = _dotT(e1, fc2W_ref[...]) + fc2b_ref[...]
    linW = linW_ref[...]
    lin_en = jnp.concatenate(
        [linW[:, 0:EMB], linW[:, EMB + _POS_EMB:EMB + _POS_EMB + 1]], axis=1)
    scene_c = _dotT(jnp.concatenate([enc_emb, na], axis=1), lin_en) \
        + linb_ref[...]

    pe = pemb_ref[...].reshape(G * A, _POS_EMB)
    h = _dotT(pe, linW[:, EMB:EMB + _POS_EMB]) + dot(Esel_ref[...], scene_c)

    pos2 = pos_ref[...].reshape(G * A, 2)
    tf = tf_ref[...].reshape(G * A, 1).astype(f32)

    iota_r = jax.lax.broadcasted_iota(jnp.int32, (G * A, W), 0)
    iota_l = jax.lax.broadcasted_iota(jnp.int32, (G * A, W), 1)
    pack_mask = (iota_r % JB) == (iota_l // EMB)
    blk_r = jax.lax.broadcasted_iota(jnp.int32, (W, W), 0)
    blk_l = jax.lax.broadcasted_iota(jnp.int32, (W, W), 1)
    blk_mask = (blk_r // EMB) == (blk_l // EMB)
    zeros_w = jnp.zeros((G * A, W), f32)
    zeros_blk = jnp.zeros((W, W), bf16)

    hx = None
    for l in range(_L):
        (m1W_ref, m1b_ref, m2W_ref, m2b_ref,
         u1W_ref, u1b_ref, u2W_ref, u2b_ref) = layer_refs[8 * l:8 * l + 8]
        m1W = m1W_ref[...]
        Wd = jnp.concatenate(
            [m1W[:, 0:EMB], -m1W[:, 2 * EMB:2 * EMB + 2],
             m1W[:, 2 * EMB + 3:2 * EMB + 4]], axis=1)
        Ws = jnp.concatenate(
            [m1W[:, EMB:2 * EMB], m1W[:, 2 * EMB:2 * EMB + 3]], axis=1)
        Wds = jnp.concatenate([Wd, Ws], axis=0)
        hx = jnp.concatenate([h, pos2, tf], axis=1)
        DS = _dotT(hx, Wds)
        D = DS[:, 0:EMB] + m1b_ref[...]
        S = DS[:, EMB:2 * EMB]
        cc = jnp.concatenate([m2W_ref[...].astype(bf16)] * JB, axis=1)
        W2blk = jnp.where(blk_mask, jnp.concatenate([cc] * JB, axis=0),
                          zeros_blk)
        b2t = jnp.concatenate([m2b_ref[...]] * JB, axis=1)
        S_masked = jnp.where(pack_mask, jnp.concatenate([S] * JB, axis=1),
                             zeros_w)
        S4 = dot(Q_ref[...], S_masked).astype(bf16) \
            .reshape(G, NJ, W)
        Dt = dot(D, TileEye_ref[...]).astype(bf16) \
            .reshape(G, A, W)
        acc = zeros_w
        for jj in range(NJ):
            pre = Dt + S4[:, jj:jj + 1, :]
            t1 = jnp.tanh(pre).reshape(G * A, W)
            acc = acc + jnp.tanh(_dotT(t1, W2blk) + b2t)
        aggr = dot(acc, F_ref[...])
        uin = jnp.concatenate([h, aggr], axis=1)
        u = jnp.tanh(_dotT(uin, u1W_ref[...]) + u1b_ref[...])
        h = h + jnp.tanh(_dotT(u, u2W_ref[...]) + u2b_ref[...])

    out_ref[...] = h.reshape(G, A, EMB)


def kernel(pos, enc, pos_emb, numAgents_emb, num_agents, T, params):
    B, A = pos.shape[0], pos.shape[1]
    L, EMB, JB = _L, _EMB, _JB
    NJ = A // JB
    f32 = jnp.float32

    G = _G
    NG = B // G
    tf = T.reshape(NG, G * A, 1)

    fc1W, fc1b = params["fc1"]
    fc2W, fc2b = params["fc2"]
    linW, linb = params["lin_in"]
    lay = params["layers"]

    layer_ops = []
    layer_specs = []

    def bs(block, imap):
        return pl.BlockSpec(block, imap)

    full2 = lambda i: (0, 0)

    for l in range(L):
        for name in ("msg1", "msg2", "upd1", "upd2"):
            Wl, bl = lay[l][name]
            layer_ops += [Wl, bl[None, :]]
            layer_specs += [bs(Wl.shape, full2), bs((1, bl.shape[0]), full2)]

    eye32 = np.eye(EMB, dtype=np.float32)
    Esel = jnp.asarray(np.kron(np.eye(G, dtype=np.float32),
                               np.ones((A, 1), np.float32)))
    TileEye = jnp.asarray(np.kron(np.ones((1, JB), np.float32), eye32))
    Q = jnp.asarray(np.kron(np.eye(G * NJ, dtype=np.float32),
                            np.ones((1, JB), np.float32)))
    F = jnp.asarray(np.kron(np.ones((JB, 1), np.float32), eye32))

    grid = (NG,)

    row2 = lambda i: (i, 0)
    row3 = lambda i: (i, 0, 0)

    in_specs = [
        bs((G, A, 2), row3),
        bs((1, G * A, 1), row3),
        bs((G, _ENC_DIM), row2),
        bs((G, A, _POS_EMB), row3),
        bs((G, 1), row2),
        bs(fc1W.shape, full2), bs((1, fc1b.shape[0]), full2),
        bs(fc2W.shape, full2), bs((1, fc2b.shape[0]), full2),
        bs(linW.shape, full2), bs((1, linb.shape[0]), full2),
    ] + layer_specs + [
        bs(Esel.shape, full2), bs(TileEye.shape, full2),
        bs(Q.shape, full2), bs(F.shape, full2),
    ]

    out = pl.pallas_call(
        _body,
        grid=grid,
        in_specs=in_specs,
        out_specs=pl.BlockSpec((G, A, EMB), row3),
        out_shape=jax.ShapeDtypeStruct((B, A, EMB), f32),
        compiler_params=pltpu.CompilerParams(
            dimension_semantics=("parallel",),
        ),
    )(pos, tf, enc, pos_emb, numAgents_emb,
      fc1W, fc1b[None, :], fc2W, fc2b[None, :], linW, linb[None, :],
      *layer_ops,
      Esel, TileEye, Q, F)
    return out


# speedup vs baseline: 1.0129x; 1.0129x over previous
import functools

import jax
import jax.numpy as jnp
import numpy as np
from jax.experimental import pallas as pl
from jax.experimental.pallas import tpu as pltpu

_B = 128
_A = 64
_EMB = 32
_POS_EMB = 16
_ENC_DIM = 128
_L = 4
_G = 16
_JB = 8


def _dotT(x, w):
    return jax.lax.dot_general(x, w, (((1,), (1,)), ((), ())),
                               preferred_element_type=jnp.float32)


def _body(*refs):
    (pos_ref, tf_ref, enc_ref, pemb_ref, na_ref,
     fc1W_ref, fc1b_ref, fc2W_ref, fc2b_ref, linW_ref, linb_ref) = refs[:11]
    layer_refs = refs[11:11 + 8 * _L]
    Esel_ref, TileEye_ref, Q_ref, F_ref, out_ref = refs[11 + 8 * _L:]

    G, A, EMB, JB = _G, _A, _EMB, _JB
    NJ = A // JB
    W = JB * EMB

    f32 = jnp.float32
    dot = functools.partial(jnp.dot, preferred_element_type=f32)

    enc = enc_ref[...]
    na = na_ref[...]
    e1 = jnp.tanh(_dotT(enc, fc1W_ref[...]) + fc1b_ref[...])
    enc_emb = _dotT(e1, fc2W_ref[...]) + fc2b_ref[...]
    linW = linW_ref[...]
    lin_en = jnp.concatenate(
        [linW[:, 0:EMB], linW[:, EMB + _POS_EMB:EMB + _POS_EMB + 1]], axis=1)
    scene_c = _dotT(jnp.concatenate([enc_emb, na], axis=1), lin_en) \
        + linb_ref[...]

    pe = pemb_ref[...].reshape(G * A, _POS_EMB)
    h = _dotT(pe, linW[:, EMB:EMB + _POS_EMB]) + dot(Esel_ref[...], scene_c)

    pos2 = pos_ref[...].reshape(G * A, 2)
    tf = tf_ref[...].reshape(G * A, 1).astype(f32)

    iota_r = jax.lax.broadcasted_iota(jnp.int32, (G * A, W), 0)
    iota_l = jax.lax.broadcasted_iota(jnp.int32, (G * A, W), 1)
    pack_mask = (iota_r % JB) == (iota_l // EMB)
    blk_r = jax.lax.broadcasted_iota(jnp.int32, (W, W), 0)
    blk_l = jax.lax.broadcasted_iota(jnp.int32, (W, W), 1)
    blk_mask = (blk_r // EMB) == (blk_l // EMB)
    zeros_w = jnp.zeros((G * A, W), f32)
    zeros_blk = jnp.zeros((W, W), f32)

    hx = None
    for l in range(_L):
        (m1W_ref, m1b_ref, m2W_ref, m2b_ref,
         u1W_ref, u1b_ref, u2W_ref, u2b_ref) = layer_refs[8 * l:8 * l + 8]
        m1W = m1W_ref[...]
        Wd = jnp.concatenate(
            [m1W[:, 0:EMB], -m1W[:, 2 * EMB:2 * EMB + 2],
             m1W[:, 2 * EMB + 3:2 * EMB + 4]], axis=1)
        Ws = jnp.concatenate(
            [m1W[:, EMB:2 * EMB], m1W[:, 2 * EMB:2 * EMB + 3]], axis=1)
        Wds = jnp.concatenate([Wd, Ws], axis=0)
        hx = jnp.concatenate([h, pos2, tf], axis=1)
        DS = _dotT(hx, Wds)
        D = DS[:, 0:EMB] + m1b_ref[...]
        S = DS[:, EMB:2 * EMB]
        cc = jnp.concatenate([m2W_ref[...]] * JB, axis=1)
        W2blk = jnp.where(blk_mask, jnp.concatenate([cc] * JB, axis=0),
                          zeros_blk)
        b2t = jnp.concatenate([m2b_ref[...]] * JB, axis=1)
        S_masked = jnp.where(pack_mask, jnp.concatenate([S] * JB, axis=1),
                             zeros_w)
        S4 = dot(Q_ref[...], S_masked).reshape(G, NJ, W)
        Dt = dot(D, TileEye_ref[...]).reshape(G, A, W)
        acc = zeros_w
        for jj in range(NJ):
            pre = Dt + S4[:, jj:jj + 1, :]
            t1 = jnp.tanh(pre).reshape(G * A, W)
            acc = acc + jnp.tanh(_dotT(t1, W2blk) + b2t)
        aggr = dot(acc, F_ref[...])
        uin = jnp.concatenate([h, aggr], axis=1)
        u = jnp.tanh(_dotT(uin, u1W_ref[...]) + u1b_ref[...])
        h = h + jnp.tanh(_dotT(u, u2W_ref[...]) + u2b_ref[...])

    out_ref[...] = h.reshape(G, A, EMB)


def kernel(pos, enc, pos_emb, numAgents_emb, num_agents, T, params):
    B, A = pos.shape[0], pos.shape[1]
    L, EMB, JB = _L, _EMB, _JB
    NJ = A // JB
    f32 = jnp.float32

    G = _G
    NG = B // G
    tf = T.reshape(NG, G * A, 1)

    fc1W, fc1b = params["fc1"]
    fc2W, fc2b = params["fc2"]
    linW, linb = params["lin_in"]
    lay = params["layers"]

    layer_ops = []
    layer_specs = []

    def bs(block, imap):
        return pl.BlockSpec(block, imap)

    full2 = lambda i: (0, 0)

    for l in range(L):
        for name in ("msg1", "msg2", "upd1", "upd2"):
            Wl, bl = lay[l][name]
            layer_ops += [Wl, bl[None, :]]
            layer_specs += [bs(Wl.shape, full2), bs((1, bl.shape[0]), full2)]

    eye32 = np.eye(EMB, dtype=np.float32)
    Esel = jnp.asarray(np.kron(np.eye(G, dtype=np.float32),
                               np.ones((A, 1), np.float32)))
    TileEye = jnp.asarray(np.kron(np.ones((1, JB), np.float32), eye32))
    Q = jnp.asarray(np.kron(np.eye(G * NJ, dtype=np.float32),
                            np.ones((1, JB), np.float32)))
    F = jnp.asarray(np.kron(np.ones((JB, 1), np.float32), eye32))

    grid = (NG,)

    row2 = lambda i: (i, 0)
    row3 = lambda i: (i, 0, 0)

    in_specs = [
        bs((G, A, 2), row3),
        bs((1, G * A, 1), row3),
        bs((G, _ENC_DIM), row2),
        bs((G, A, _POS_EMB), row3),
        bs((G, 1), row2),
        bs(fc1W.shape, full2), bs((1, fc1b.shape[0]), full2),
        bs(fc2W.shape, full2), bs((1, fc2b.shape[0]), full2),
        bs(linW.shape, full2), bs((1, linb.shape[0]), full2),
    ] + layer_specs + [
        bs(Esel.shape, full2), bs(TileEye.shape, full2),
        bs(Q.shape, full2), bs(F.shape, full2),
    ]

    out = pl.pallas_call(
        _body,
        grid=grid,
        in_specs=in_specs,
        out_specs=pl.BlockSpec((G, A, EMB), row3),
        out_shape=jax.ShapeDtypeStruct((B, A, EMB), f32),
        compiler_params=pltpu.CompilerParams(
            dimension_semantics=("parallel",),
        ),
    )(pos, tf, enc, pos_emb, numAgents_emb,
      fc1W, fc1b[None, :], fc2W, fc2b[None, :], linW, linb[None, :],
      *layer_ops,
      Esel, TileEye, Q, F)
    return out
